# baseline (speedup 1.0000x reference)
import functools

import jax
import jax.numpy as jnp
from jax import lax
from jax.experimental import pallas as pl
from jax.experimental.pallas import tpu as pltpu
from jax.experimental.pallas import tpu_sc as plsc

_DIM = 64
_BATCH = 16384
_NNEG = 20
_NPAIR = _NNEG + 1
_NW = 32
_CH = 128
_NSLICE = 4
_BS = _BATCH // _NSLICE

_LOG2E = 1.4426950408889634

_POLY = (-0.3665167014303693, 0.7213459840780102, -0.07976529329011446,
         -0.004957223416335807, 0.002184405606105031, 0.00022657838744066794)


def _f_poly(t):
    acc = jnp.full_like(t, _POLY[-1])
    for c in _POLY[-2::-1]:
        acc = acc * t + c
    return acc


def _sigmoid(x):
    return 1.0 / (1.0 + jnp.exp2(x * -_LOG2E))


_GCH = 3
_NGRP = 7


def _sc_gather_body(w_word, w_ctx, idx_u, idx_c, out_u, out_c,
                    idx_uv, idx_cv, u_buf, rows0, rows1, *sems):
    gsems = sems[0:2 * _GCH]
    osems = sems[2 * _GCH:2 * _GCH + 2]
    usems = sems[2 * _GCH + 2:]
    wid = lax.axis_index("s") * 2 + lax.axis_index("c")
    u_per_w = _BS // _NW
    c_per_w = _NPAIR * _BS // _NW
    pltpu.sync_copy(idx_u.at[pl.ds(wid * u_per_w, u_per_w)], idx_uv)
    pltpu.sync_copy(idx_c.at[pl.ds(wid * c_per_w, c_per_w)], idx_cv)
    pltpu.async_copy(w_word.at[idx_uv], u_buf, usems[0]).wait()
    u_out = pltpu.async_copy(u_buf, out_u.at[pl.ds(wid * u_per_w, u_per_w)],
                             usems[1])
    bufs = (rows0, rows1)
    outs = [None, None]
    for g in range(_NGRP):
        sb = g & 1
        if outs[sb] is not None:
            outs[sb].wait()
        gathers = []
        for t in range(_GCH):
            ch = g * _GCH + t
            gathers.append(pltpu.async_copy(
                w_ctx.at[idx_cv.at[pl.ds(ch * _CH, _CH)]],
                bufs[sb].at[pl.ds(t * _CH, _CH)], gsems[sb * _GCH + t]))
        for t in range(_GCH):
            gathers[t].wait()
        outs[sb] = pltpu.async_copy(
            bufs[sb],
            out_c.at[pl.ds(wid * c_per_w + g * _GCH * _CH, _GCH * _CH)],
            osems[sb])
    outs[0].wait()
    outs[1].wait()
    u_out.wait()


@functools.cache
def _sc_gather():
    return pl.kernel(
        _sc_gather_body,
        out_type=(
            jax.ShapeDtypeStruct((_BS, 2 * _DIM), jnp.float32),
            jax.ShapeDtypeStruct((_NPAIR * _BS, 2 * _DIM), jnp.float32),
        ),
        mesh=plsc.VectorSubcoreMesh(core_axis_name="c", subcore_axis_name="s"),
        scratch_types=(
            [pltpu.VMEM((_BS // _NW,), jnp.int32),
             pltpu.VMEM((_NPAIR * _BS // _NW,), jnp.int32),
             pltpu.VMEM((_BS // _NW, 2 * _DIM), jnp.float32),
             pltpu.VMEM((_GCH * _CH, 2 * _DIM), jnp.float32),
             pltpu.VMEM((_GCH * _CH, 2 * _DIM), jnp.float32)]
            + [pltpu.SemaphoreType.DMA for _ in range(2 * _GCH + 4)]
        ),
    )


def _box_t(x):
    s = _sigmoid(x).T
    z = s[:_DIM]
    Z = z + s[_DIM:] * (1.0 - z)
    return z, Z


def _tc_body(u_ref, c_ref, vols_ref, ints_ref, tv_ref, zZu_ref):
    j = pl.program_id(1)

    @pl.when(j == 0)
    def _():
        zu0, Zu0 = _box_t(u_ref[...])
        zZu_ref[:_DIM] = zu0
        zZu_ref[_DIM:] = Zu0
        tv_ref[0, 0, :] = jnp.sum(_f_poly(Zu0 - zu0), axis=0)

    zu = zZu_ref[:_DIM]
    Zu = zZu_ref[_DIM:]
    zc, Zc = _box_t(c_ref[...])
    fv = _f_poly(Zc - zc)
    fi = _f_poly(jnp.minimum(Zc, Zu) - jnp.maximum(zc, zu))
    vols_ref[0, 0, :] = jnp.sum(fv, axis=0)
    ints_ref[0, 0, :] = jnp.sum(fi, axis=0)


def _tc_compute(u_rows, ctx_rows, bb=1024):
    nb = _BS // bb
    return pl.pallas_call(
        _tc_body,
        grid=(nb, _NPAIR),
        in_specs=[
            pl.BlockSpec((bb, 2 * _DIM), lambda i, j: (i, 0)),
            pl.BlockSpec((bb, 2 * _DIM), lambda i, j, nb=nb: (j * nb + i, 0)),
        ],
        out_specs=[
            pl.BlockSpec((1, 1, bb), lambda i, j: (j, 0, i)),
            pl.BlockSpec((1, 1, bb), lambda i, j: (j, 0, i)),
            pl.BlockSpec((1, 1, bb), lambda i, j: (0, 0, i)),
        ],
        out_shape=[
            jax.ShapeDtypeStruct((_NPAIR, 1, _BS), jnp.float32),
            jax.ShapeDtypeStruct((_NPAIR, 1, _BS), jnp.float32),
            jax.ShapeDtypeStruct((1, 1, _BS), jnp.float32),
        ],
        scratch_shapes=[pltpu.VMEM((2 * _DIM, bb), jnp.float32)],
    )(u_rows, ctx_rows)


def kernel(pos_u, pos_w, neg_w, W_word, W_ctx):
    pos_u = pos_u.astype(jnp.int32)
    pos_w = pos_w.astype(jnp.int32)
    neg_w = neg_w.astype(jnp.int32)
    vols_l, ints_l, tv_l = [], [], []
    for k in range(_NSLICE):
        sl = slice(k * _BS, (k + 1) * _BS)
        idx_ctx = jnp.concatenate([neg_w[sl].T.reshape(-1), pos_w[sl]])
        u_rows, ctx_rows = _sc_gather()(W_word, W_ctx, pos_u[sl], idx_ctx)
        vols, ints, tv = _tc_compute(u_rows, ctx_rows)
        vols_l.append(vols[:, 0, :])
        ints_l.append(ints[:, 0, :])
        tv_l.append(tv[0, 0, :])
    vols = jnp.concatenate(vols_l, axis=1)
    ints = jnp.concatenate(ints_l, axis=1)
    tv = jnp.concatenate(tv_l)
    return (tv, vols[_NNEG], vols[:_NNEG].T, ints[_NNEG], ints[:_NNEG].T)
